# baseline (speedup 1.0000x reference)
import functools
import math

import jax
import jax.numpy as jnp
from jax import lax
from jax.experimental import pallas as pl
from jax.experimental.pallas import tpu as pltpu
from jax.experimental.pallas import tpu_sc as plsc

_SMOOTHING = 0.1
_PAD = 0

_NC = 2
_NS = 16


def _sc_gather_body(xflat_hbm, tgt_hbm, out_hbm, tgt_v, idx_v, vals_v, res_v,
                    sem, *, v, rows_per_worker):
    wid = lax.axis_index("s") * _NC + lax.axis_index("c")
    base = wid * rows_per_worker
    pltpu.sync_copy(tgt_hbm.at[pl.ds(base, rows_per_worker)], tgt_v)
    nchunks = rows_per_worker // 16
    for c in range(nchunks):
        t16 = tgt_v[pl.ds(c * 16, 16)]
        rows16 = (base + c * 16 + lax.iota(jnp.int32, 16)) * v
        idx_v[pl.ds(c * 16, 16)] = rows16 + t16
        idx_v[pl.ds(rows_per_worker + c * 16, 16)] = rows16
    pltpu.async_copy(xflat_hbm.at[idx_v], vals_v, sem).wait()
    zero = jnp.zeros((16,), jnp.float32)
    st = zero
    s0 = zero
    nv = zero
    for c in range(nchunks):
        valid = tgt_v[pl.ds(c * 16, 16)] != _PAD
        st = st + jnp.where(valid, vals_v[pl.ds(c * 16, 16)], 0.0)
        s0 = s0 + jnp.where(valid, vals_v[pl.ds(rows_per_worker + c * 16, 16)], 0.0)
        nv = nv + jnp.where(valid, 1.0, 0.0)
    res_v[pl.ds(0, 16)] = st
    res_v[pl.ds(16, 16)] = s0
    res_v[pl.ds(32, 16)] = nv
    pltpu.sync_copy(res_v, out_hbm.at[wid])


def _sc_gather(x_flat, tgt, v, n):
    rows_per_worker = n // (_NC * _NS)
    mesh = plsc.VectorSubcoreMesh(core_axis_name="c", subcore_axis_name="s",
                                  num_cores=_NC, num_subcores=_NS)
    kfn = pl.kernel(
        functools.partial(_sc_gather_body, v=v, rows_per_worker=rows_per_worker),
        out_type=jax.ShapeDtypeStruct((_NC * _NS, 48), jnp.float32),
        mesh=mesh,
        scratch_types=[
            pltpu.VMEM((rows_per_worker,), jnp.int32),
            pltpu.VMEM((2 * rows_per_worker,), jnp.int32),
            pltpu.VMEM((2 * rows_per_worker,), jnp.float32),
            pltpu.VMEM((48,), jnp.float32),
            pltpu.SemaphoreType.DMA,
        ],
    )
    return kfn(x_flat, tgt)


def _bulk_body(tgt_ref, xf_ref, sc_ref, out_ref, acc_ref, *, eps, cval, v,
               rows_per_blk):
    k = pl.program_id(0)
    xb = xf_ref[...]
    s = jnp.sum(xb)
    tb = tgt_ref[pl.ds(k * rows_per_blk, rows_per_blk), :]
    has_pad = jnp.any(tb == _PAD)

    @pl.when(k == 0)
    def _():
        acc_ref[0] = 0.0

    @pl.when(jnp.logical_not(has_pad))
    def _():
        acc_ref[0] += s

    @pl.when(has_pad)
    def _():
        e = (lax.broadcasted_iota(jnp.int32, xb.shape, 0) * 128
             + lax.broadcasted_iota(jnp.int32, xb.shape, 1))
        corr = 0.0
        for j in range(rows_per_blk):
            in_row = jnp.logical_and(e >= j * v, e < (j + 1) * v)
            wj = jnp.where(tb[j, 0] == _PAD, 1.0, 0.0)
            corr = corr + wj * jnp.sum(jnp.where(in_row, xb, 0.0))
        acc_ref[0] += s - corr

    @pl.when(k == pl.num_programs(0) - 1)
    def _():
        scp = sc_ref[...]
        st = jnp.sum(scp[:, 0:16])
        s0 = jnp.sum(scp[:, 16:32])
        nv = jnp.sum(scp[:, 32:48])
        out_ref[0, 0] = (-eps) * acc_ref[0] + eps * s0 \
            + (eps - (1.0 - _SMOOTHING)) * st + cval * nv


def kernel(x, target):
    n, v = x.shape
    eps = _SMOOTHING / (v - 2)
    cval = _SMOOTHING * math.log(eps) + (1.0 - _SMOOTHING) * math.log(1.0 - _SMOOTHING)
    tgt = target.astype(jnp.int32)
    x_flat = x.reshape(-1)
    sc_out = _sc_gather(x_flat, tgt, v, n)

    rows_per_blk = 32
    nblk = n // rows_per_blk
    blk_flat_rows = rows_per_blk * v // 128
    xf = x_flat.reshape(n * v // 128, 128)
    tgt2d = tgt.reshape(n, 1)
    out = pl.pallas_call(
        functools.partial(_bulk_body, eps=eps, cval=cval, v=v,
                          rows_per_blk=rows_per_blk),
        grid=(nblk,),
        in_specs=[
            pl.BlockSpec((n, 1), lambda k: (0, 0)),
            pl.BlockSpec((blk_flat_rows, 128), lambda k: (k, 0)),
            pl.BlockSpec((_NC * _NS, 48), lambda k: (0, 0)),
        ],
        out_specs=pl.BlockSpec(memory_space=pltpu.SMEM),
        out_shape=jax.ShapeDtypeStruct((1, 1), jnp.float32),
        scratch_shapes=[pltpu.SMEM((1,), jnp.float32)],
        compiler_params=pltpu.CompilerParams(
            dimension_semantics=("arbitrary",),
        ),
    )(tgt2d, xf, sc_out)
    return out.reshape(())


# speedup vs baseline: 3.1474x; 3.1474x over previous
import functools
import math

import jax
import jax.numpy as jnp
from jax.experimental import pallas as pl
from jax.experimental.pallas import tpu as pltpu

_SMOOTHING = 0.1
_PAD = 0


def _body(tgt_ref, x_ref, out_ref, acc_ref, *, eps, cval):
    k = pl.program_id(0)
    br = x_ref.shape[0]
    tgt = tgt_ref[pl.ds(k * br, br), :]
    vf = (tgt != _PAD).astype(jnp.float32)
    xb = x_ref[...]
    xv = xb * vf
    s_all = jnp.sum(xv)
    s_col0 = jnp.sum(xv[:, 0])
    cols = jax.lax.broadcasted_iota(jnp.int32, xb.shape, 1)
    s_tgt = jnp.sum(jnp.where(cols == tgt, xv, 0.0))
    partial = (-eps) * s_all + eps * s_col0 + (eps - (1.0 - _SMOOTHING)) * s_tgt \
        + cval * jnp.sum(vf)

    @pl.when(k == 0)
    def _():
        acc_ref[0] = 0.0

    acc_ref[0] += partial

    @pl.when(k == pl.num_programs(0) - 1)
    def _():
        out_ref[0, 0] = acc_ref[0]


def kernel(x, target):
    n, v = x.shape
    eps = _SMOOTHING / (v - 2)
    cval = _SMOOTHING * math.log(eps) + (1.0 - _SMOOTHING) * math.log(1.0 - _SMOOTHING)
    br = 32 if n % 32 == 0 else 1
    grid = n // br
    tgt2d = target.astype(jnp.int32).reshape(n, 1)
    out = pl.pallas_call(
        functools.partial(_body, eps=eps, cval=cval),
        grid=(grid,),
        in_specs=[
            pl.BlockSpec((n, 1), lambda k: (0, 0)),
            pl.BlockSpec((br, v), lambda k: (k, 0)),
        ],
        out_specs=pl.BlockSpec(memory_space=pltpu.SMEM),
        out_shape=jax.ShapeDtypeStruct((1, 1), jnp.float32),
        scratch_shapes=[pltpu.SMEM((1,), jnp.float32)],
        compiler_params=pltpu.CompilerParams(
            dimension_semantics=("arbitrary",),
        ),
    )(tgt2d, x)
    return out.reshape(())


# speedup vs baseline: 3.3356x; 1.0598x over previous
import functools
import math

import jax
import jax.numpy as jnp
from jax.experimental import pallas as pl
from jax.experimental.pallas import tpu as pltpu

_SMOOTHING = 0.1
_PAD = 0
_M = 4
_BR = 8


def _body(tgt_ref, *refs, eps, cval):
    x_refs = refs[:_M]
    out_ref, acc_ref = refs[_M], refs[_M + 1]
    k = pl.program_id(0)
    grid = pl.num_programs(0)

    @pl.when(k == 0)
    def _():
        acc_ref[0] = 0.0

    partial = 0.0
    for i, x_ref in enumerate(x_refs):
        br = x_ref.shape[0]
        row0 = (i * grid + k) * br
        tgt = tgt_ref[pl.ds(row0, br), :]
        vf = (tgt != _PAD).astype(jnp.float32)[:, 0]
        xb = x_ref[...]
        rowsum = jnp.sum(xb, axis=1)
        cols = jax.lax.broadcasted_iota(jnp.int32, xb.shape, 1)
        tgtv = jnp.sum(jnp.where(cols == tgt, xb, 0.0), axis=1)
        col0 = xb[:, 0]
        per_row = (-eps) * rowsum + eps * col0 \
            + (eps - (1.0 - _SMOOTHING)) * tgtv + cval
        partial += jnp.sum(vf * per_row)

    acc_ref[0] += partial

    @pl.when(k == pl.num_programs(0) - 1)
    def _():
        out_ref[0, 0] = acc_ref[0]


def kernel(x, target):
    n, v = x.shape
    eps = _SMOOTHING / (v - 2)
    cval = _SMOOTHING * math.log(eps) + (1.0 - _SMOOTHING) * math.log(1.0 - _SMOOTHING)
    if n % (_M * _BR) == 0:
        m, br = _M, _BR
    else:
        m, br = 1, 1
    grid = n // (m * br)
    tgt2d = target.astype(jnp.int32).reshape(n, 1)

    def mk_spec(i):
        return pl.BlockSpec((br, v), lambda k, i=i: (i * grid + k, 0))

    out = pl.pallas_call(
        functools.partial(_body, eps=eps, cval=cval),
        grid=(grid,),
        in_specs=[pl.BlockSpec((n, 1), lambda k: (0, 0))]
        + [mk_spec(i) for i in range(m)],
        out_specs=pl.BlockSpec(memory_space=pltpu.SMEM),
        out_shape=jax.ShapeDtypeStruct((1, 1), jnp.float32),
        scratch_shapes=[pltpu.SMEM((1,), jnp.float32)],
        compiler_params=pltpu.CompilerParams(
            dimension_semantics=("arbitrary",),
        ),
    )(tgt2d, *([x] * m))
    return out.reshape(())
